# baseline (speedup 1.0000x reference)
import jax
import jax.numpy as jnp
from jax import lax
from jax.experimental import pallas as pl
from jax.experimental.pallas import tpu as pltpu
from jax.experimental.pallas import tpu_sc as plsc

VOCAB_E = 100000
EMBED_E = 128
B_E = 4096
L_E = 50

NC = 2
NS = 16
NW = NC * NS

N_TOK = B_E * L_E
PER_W = N_TOK // NW
CH = 256
NCHUNK = PER_W // CH
ROW_CH = B_E // CH


def _emb_body(tokens_hbm, table_hbm, out_hbm, idx0, idx1, idx2, rows0, rows1,
              rows2, isem0, isem1, isem2, gsem0, gsem1, gsem2, osem0, osem1,
              osem2):
  wid = lax.axis_index("s") * NC + lax.axis_index("c")
  gc0 = wid * NCHUNK

  idx = (idx0, idx1, idx2)
  rows = (rows0, rows1, rows2)
  isem = (isem0, isem1, isem2)
  gsem = (gsem0, gsem1, gsem2)
  osem = (osem0, osem1, osem2)

  def load_idx(b, c):
    gc = jnp.minimum(gc0 + c, N_TOK // CH - 1)
    pltpu.async_copy(tokens_hbm.at[gc // ROW_CH, pl.ds((gc % ROW_CH) * CH, CH)],
                     idx[b], isem[b])

  def wait_idx(b):
    pltpu.make_async_copy(tokens_hbm.at[0, pl.ds(0, CH)], idx[b],
                          isem[b]).wait()

  def wait_write(b):
    pltpu.make_async_copy(rows[b], out_hbm.at[pl.ds(0, CH)], osem[b]).wait()

  def wait_gather(b):
    pltpu.make_async_copy(table_hbm.at[idx[b]], rows[b], gsem[b]).wait()

  def gather(b):
    pltpu.async_copy(table_hbm.at[idx[b]], rows[b], gsem[b])

  def write_out(b, c):
    pltpu.async_copy(rows[b], out_hbm.at[pl.ds((gc0 + c) * CH, CH)], osem[b])

  def write_back(b, c):
    wait_gather(b)
    write_out(b, c)
    load_idx(b, c + 3)

  load_idx(0, 0)
  load_idx(1, 1)
  load_idx(2, 2)
  wait_idx(0)
  gather(0)
  wait_idx(1)
  gather(1)
  write_back(0, 0)
  wait_idx(2)
  gather(2)
  write_back(1, 1)

  def triple(k, carry):
    c = 3 * k
    wait_idx(0)
    wait_write(0)
    gather(0)
    write_back(2, c - 1)
    wait_idx(1)
    wait_write(1)
    gather(1)
    write_back(0, c)
    wait_idx(2)
    wait_write(2)
    gather(2)
    write_back(1, c + 1)
    return carry

  lax.fori_loop(1, NCHUNK // 3, triple, 0, unroll=False)

  cl = NCHUNK - 1
  wait_idx(0)
  wait_write(0)
  gather(0)
  wait_gather(2)
  write_out(2, cl - 1)
  wait_gather(0)
  write_out(0, cl)
  wait_idx(1)
  wait_write(0)
  wait_write(1)
  wait_write(2)


@jax.jit
def _embed(tokens_t, table):
  k = pl.kernel(
      _emb_body,
      out_type=jax.ShapeDtypeStruct((N_TOK, EMBED_E), jnp.float32),
      mesh=plsc.VectorSubcoreMesh(core_axis_name="c", subcore_axis_name="s"),
      scratch_types=(
          [pltpu.VMEM((CH,), jnp.int32) for _ in range(3)]
          + [pltpu.VMEM((CH, EMBED_E), jnp.float32) for _ in range(3)]
          + [pltpu.SemaphoreType.DMA] * 9
      ),
  )
  return k(tokens_t, table)


def kernel(tokens, table):
  tokens_t = jnp.transpose(tokens).astype(jnp.int32)
  out = _embed(tokens_t, table)
  return jnp.transpose(out.reshape(L_E, B_E, EMBED_E), (1, 0, 2))


# speedup vs baseline: 1.0091x; 1.0091x over previous
import jax
import jax.numpy as jnp
from jax import lax
from jax.experimental import pallas as pl
from jax.experimental.pallas import tpu as pltpu
from jax.experimental.pallas import tpu_sc as plsc

VOCAB_E = 100000
EMBED_E = 128
B_E = 4096
L_E = 50

NC = 2
NS = 16
NW = NC * NS

N_TOK = B_E * L_E
PER_W = N_TOK // NW
CH = 256
NCHUNK = PER_W // CH
ROW_CH = B_E // CH


def _emb_body(tokens_hbm, table_hbm, out_hbm, idx0, idx1, rows0, rows1,
              isem0, isem1, gsem0, gsem1, osem0, osem1):
  wid = lax.axis_index("s") * NC + lax.axis_index("c")
  gc0 = wid * NCHUNK

  idx = (idx0, idx1)
  rows = (rows0, rows1)
  isem = (isem0, isem1)
  gsem = (gsem0, gsem1)
  osem = (osem0, osem1)

  def load_idx(b, c):
    gc = gc0 + c
    pltpu.async_copy(tokens_hbm.at[gc // ROW_CH, pl.ds((gc % ROW_CH) * CH, CH)],
                     idx[b], isem[b])

  def wait_idx(b):
    pltpu.make_async_copy(tokens_hbm.at[0, pl.ds(0, CH)], idx[b],
                          isem[b]).wait()

  def wait_write(b):
    pltpu.make_async_copy(rows[b], out_hbm.at[pl.ds(0, CH)], osem[b]).wait()

  def wait_gather(b):
    pltpu.make_async_copy(table_hbm.at[idx[b]], rows[b], gsem[b]).wait()

  def gather(b):
    pltpu.async_copy(table_hbm.at[idx[b]], rows[b], gsem[b])

  def write_out(b, c):
    pltpu.async_copy(rows[b], out_hbm.at[pl.ds((gc0 + c) * CH, CH)], osem[b])

  def write_back(b, c):
    wait_gather(b)
    write_out(b, c)
    load_idx(b, c + 2)

  load_idx(0, 0)
  load_idx(1, 1)
  wait_idx(0)
  gather(0)
  wait_idx(1)
  gather(1)
  write_back(0, 0)

  def pair(k, carry):
    c = 2 * k
    wait_idx(0)
    wait_write(0)
    gather(0)
    write_back(1, c - 1)
    wait_idx(1)
    wait_write(1)
    gather(1)
    write_back(0, c)
    return carry

  lax.fori_loop(1, NCHUNK // 2, pair, 0, unroll=False)

  cl = NCHUNK - 1
  wait_idx(0)
  wait_write(0)
  gather(0)
  wait_gather(1)
  write_out(1, cl - 1)
  wait_gather(0)
  write_out(0, cl)
  wait_write(0)
  wait_write(1)


@jax.jit
def _embed(tokens_t, table):
  k = pl.kernel(
      _emb_body,
      out_type=jax.ShapeDtypeStruct((N_TOK, EMBED_E), jnp.float32),
      mesh=plsc.VectorSubcoreMesh(core_axis_name="c", subcore_axis_name="s"),
      scratch_types=(
          [pltpu.VMEM((CH,), jnp.int32) for _ in range(2)]
          + [pltpu.VMEM((CH, EMBED_E), jnp.float32) for _ in range(2)]
          + [pltpu.SemaphoreType.DMA] * 6
      ),
  )
  return k(tokens_t, table)


def kernel(tokens, table):
  tokens_t = jnp.transpose(tokens).astype(jnp.int32)
  out = _embed(tokens_t, table)
  return jnp.transpose(out.reshape(L_E, B_E, EMBED_E), (1, 0, 2))
